# baseline (speedup 1.0000x reference)
import jax
import jax.numpy as jnp
from jax.experimental import pallas as pl
from jax.experimental.pallas import tpu as pltpu

_DELTA_V = 0.5
_DELTA_D = 1.5
_GAMMA = 0.001
_C = 24
_CL = 19
_TILE = 512

_F32 = jnp.float32
_PREC = jax.lax.Precision.HIGHEST


def _dot(a, b, dims, precision=_PREC):
    return jax.lax.dot_general(a, b, (dims, ((), ())), precision=precision,
                               preferred_element_type=_F32)


def _loss_kernel(emb_t_ref, pts_t_ref, lab_ref, out_ref,
                 emb_s, en2T_s, labT_s):
    embr = emb_t_ref[0]
    pts = pts_t_ref[0]
    lab = lab_ref[0]
    D, N = embr.shape
    NT = N // _TILE

    nrm = jnp.sqrt(jnp.sum(embr * embr, axis=0, keepdims=True))
    emb = embr / jnp.maximum(nrm, 1e-12)
    en2 = jnp.sum(emb * emb, axis=0, keepdims=True)
    na = jnp.sqrt(en2)

    cid = jax.lax.broadcasted_iota(jnp.int32, (_C, N), 0) + 1
    masks = (lab == cid).astype(_F32)
    counts = jnp.sum(masks, axis=1, keepdims=True)
    present = (counts > 0).astype(_F32)
    ni = jnp.sum(present)
    denom = jnp.maximum(counts, 1.0)

    sum_emb = _dot(masks, emb, (((1,), (1,))))
    fc = sum_emb / denom
    centers = fc * present
    cnorm = jnp.sqrt(jnp.sum(centers * centers, axis=1, keepdims=True))
    cn = centers / jnp.maximum(cnorm, 1e-12)

    ec = _dot(cn, emb, (((1,), (0,))),
              precision=jax.lax.Precision.DEFAULT)
    cn2 = jnp.sum(cn * cn, axis=1, keepdims=True)
    d = jnp.sqrt(jnp.maximum(en2 + cn2 - 2.0 * ec, 0.0))
    vt = jnp.sum(masks * jnp.maximum(d - _DELTA_V, 0.0), axis=1,
                 keepdims=True) / denom
    var_b = jnp.sum(vt * present)

    cc = _dot(cn, cn, (((1,), (1,))))
    cd = jnp.sqrt(jnp.maximum(cn2 + jnp.transpose(cn2) - 2.0 * cc, 1e-12))
    eye = (jax.lax.broadcasted_iota(jnp.int32, (_C, _C), 0) ==
           jax.lax.broadcasted_iota(jnp.int32, (_C, _C), 1)).astype(_F32)
    pairmask = present * jnp.transpose(present) * (1.0 - eye)
    dist_num = jnp.sum(pairmask * jnp.maximum(2.0 * _DELTA_D - cd, 0.0))
    dist_b = dist_num / jnp.maximum(ni * (ni - 1.0), 1.0)

    reg_b = _GAMMA * jnp.sqrt(jnp.sum((cn * present) ** 2))

    sum_pts = _dot(masks, pts, (((1,), (1,))))
    gc = sum_pts / denom
    gp = _dot(gc, pts, (((1,), (0,))),
              precision=jax.lax.Precision.DEFAULT)
    p2 = jnp.sum(pts * pts, axis=0, keepdims=True)
    g2 = jnp.sum(gc * gc, axis=1, keepdims=True)
    sd = jnp.sqrt(jnp.maximum(p2 + g2 - 2.0 * gp, 0.0))
    fe = _dot(fc, emb, (((1,), (0,))),
              precision=jax.lax.Precision.DEFAULT)
    f2 = jnp.sum(fc * fc, axis=1, keepdims=True)
    fd = jnp.sqrt(jnp.maximum(en2 + f2 - 2.0 * fe, 0.0))
    ct = jnp.sum(masks * sd * fd, axis=1, keepdims=True) / denom
    center_b = jnp.sum(ct * present)

    mn = jnp.sqrt(jnp.sum(fc * fc, axis=1, keepdims=True))
    m_unit = fc / jnp.maximum(mn, 1e-12)
    mu_norm = jnp.sqrt(jnp.sum(m_unit * m_unit, axis=1, keepdims=True))
    sim = _dot(m_unit, emb, (((1,), (0,))))
    sim = sim / jnp.maximum(na * mu_norm, 1e-8)
    sim = (sim + 1.0) * 0.5

    emb_s[...] = emb
    en2T_s[...] = jnp.transpose(en2)
    labT_s[...] = jnp.transpose(lab)

    def bbody(t, carry):
        pos_s, inst_s, dm_s = carry
        j0 = t * _TILE
        etile = emb_s[:, pl.ds(j0, _TILE)]
        dots = _dot(etile, emb, (((0,), (0,))),
                    precision=jax.lax.Precision.DEFAULT)
        ej2 = en2T_s[pl.ds(j0, _TILE), :]
        dm = jnp.sqrt(jnp.maximum(ej2 + en2 - 2.0 * dots, 1e-12))
        labj = labT_s[pl.ds(j0, _TILE), :]
        inst = (labj == lab).astype(_F32)
        pos_s = pos_s + jnp.sum(dm * inst)
        inst_s = inst_s + jnp.sum(inst)
        dm_s = dm_s + jnp.sum(dm)
        return pos_s, inst_s, dm_s

    zero = _F32(0.0)
    pos_s, inst_s, dm_s = jax.lax.fori_loop(0, NT, bbody,
                                            (zero, zero, zero))
    bnd_cnt = _F32(N) * _F32(N) - inst_s
    neg_s = 2.0 * _DELTA_D * bnd_cnt - (dm_s - pos_s)
    boundary_b = (pos_s / jnp.maximum(inst_s, 1.0)
                  + neg_s / jnp.maximum(bnd_cnt, 1.0))

    lane = jax.lax.broadcasted_iota(jnp.int32, (1, N), 1)
    keys = (jax.lax.shift_left(
        jax.lax.bitcast_convert_type(jnp.maximum(sim, 0.0), jnp.int32), 1)
        | (lab == cid).astype(jnp.int32))
    k = 2
    while k <= N:
        desc = (lane & k) == 0
        j = k // 2
        while j >= 1:
            lo = (lane & j) == 0
            pk = jnp.where(lo, jnp.roll(keys, -j, axis=1),
                           jnp.roll(keys, j, axis=1))
            take_max = lo == desc
            keep = take_max == (keys > pk)
            keys = jnp.where(keep, keys, pk)
            j //= 2
        k *= 2
    payl = (keys & 1).astype(_F32)
    keys = jax.lax.bitcast_convert_type(
        jax.lax.shift_right_logical(keys, 1), _F32)

    cs = payl
    s = 1
    while s < N:
        cs = cs + jnp.where(lane >= s, jnp.roll(cs, s, axis=1), 0.0)
        s *= 2

    r = (lane + 1).astype(_F32)
    G = counts
    jac = 1.0 - (G - cs) / (G + r - cs + 1e-6)
    jacp = 1.0 - (G - cs + payl) / (G + r - 1.0 - cs + payl + 1e-6)
    lov_t = jnp.sum(jnp.maximum(keys, 0.0) * (jac - jacp), axis=1,
                    keepdims=True)
    lov_b = jnp.sum(lov_t * present) / jnp.maximum(ni, 1.0)

    vals = [var_b, dist_b, reg_b, center_b, boundary_b, lov_b, zero, zero]
    out_ref[0] = jnp.concatenate(
        [jnp.full((1, 128), v, _F32) for v in vals], axis=0)


def kernel(points, embeddings, instance_labels):
    B, N, _ = points.shape
    pts_t = jnp.moveaxis(jnp.pad(points, ((0, 0), (0, 0), (0, 5))), 2, 1)
    emb_t = jnp.moveaxis(embeddings, 2, 1)
    lab3 = instance_labels[:, None, :]

    out = pl.pallas_call(
        _loss_kernel,
        grid=(B,),
        in_specs=[
            pl.BlockSpec((1, emb_t.shape[1], N), lambda b: (b, 0, 0)),
            pl.BlockSpec((1, 8, N), lambda b: (b, 0, 0)),
            pl.BlockSpec((1, 1, N), lambda b: (b, 0, 0)),
        ],
        out_specs=pl.BlockSpec((1, 8, 128), lambda b: (b, 0, 0)),
        out_shape=jax.ShapeDtypeStruct((B, 8, 128), jnp.float32),
        compiler_params=pltpu.CompilerParams(
            dimension_semantics=("parallel",)),
        scratch_shapes=[
            pltpu.VMEM((emb_t.shape[1], N), jnp.float32),
            pltpu.VMEM((N, 1), jnp.float32),
            pltpu.VMEM((N, 1), jnp.int32),
        ],
    )(emb_t, pts_t, lab3)

    v = out[:, :, 0]
    var_loss = jnp.sum(v[:, 0]) / (B + 1e-6)
    dist_loss = jnp.sum(v[:, 1]) / (B + 1e-6)
    reg_loss = jnp.sum(v[:, 2]) / (B + 1e-6)
    center_loss = jnp.sum(v[:, 3]) / B
    boundary_loss = jnp.sum(v[:, 4]) / B
    lovasz_loss = jnp.sum(v[:, 5]) / B
    total = (0.1 * (var_loss + dist_loss + reg_loss)
             + 0.1 * center_loss
             + 0.05 * boundary_loss
             + 0.01 * lovasz_loss)
    return (total, var_loss, dist_loss, reg_loss, center_loss,
            boundary_loss, lovasz_loss)


# speedup vs baseline: 1.1617x; 1.1617x over previous
import jax
import jax.numpy as jnp
from jax.experimental import pallas as pl
from jax.experimental.pallas import tpu as pltpu

_DELTA_V = 0.5
_DELTA_D = 1.5
_GAMMA = 0.001
_C = 24
_CL = 19
_TILE = 512

_F32 = jnp.float32
_PREC = jax.lax.Precision.HIGHEST


def _dot(a, b, dims, precision=_PREC):
    return jax.lax.dot_general(a, b, (dims, ((), ())), precision=precision,
                               preferred_element_type=_F32)


def _loss_kernel(emb_t_ref, pts_t_ref, lab_ref, out_ref,
                 emb_s, en2_s, en2T_s, labT_s):
    embr = emb_t_ref[0]
    pts = pts_t_ref[0]
    lab = lab_ref[0]
    D, N = embr.shape
    NT = N // _TILE

    nrm = jnp.sqrt(jnp.sum(embr * embr, axis=0, keepdims=True))
    emb = embr / jnp.maximum(nrm, 1e-12)
    en2 = jnp.sum(emb * emb, axis=0, keepdims=True)
    na = jnp.sqrt(en2)

    cid = jax.lax.broadcasted_iota(jnp.int32, (_C, N), 0) + 1
    masks = (lab == cid).astype(_F32)
    counts = jnp.sum(masks, axis=1, keepdims=True)
    present = (counts > 0).astype(_F32)
    ni = jnp.sum(present)
    denom = jnp.maximum(counts, 1.0)

    sum_emb = _dot(masks, emb, (((1,), (1,))))
    fc = sum_emb / denom
    centers = fc * present
    cnorm = jnp.sqrt(jnp.sum(centers * centers, axis=1, keepdims=True))
    cn = centers / jnp.maximum(cnorm, 1e-12)

    ec = _dot(cn, emb, (((1,), (0,))),
              precision=jax.lax.Precision.DEFAULT)
    cn2 = jnp.sum(cn * cn, axis=1, keepdims=True)
    d = jnp.sqrt(jnp.maximum(en2 + cn2 - 2.0 * ec, 0.0))
    vt = jnp.sum(masks * jnp.maximum(d - _DELTA_V, 0.0), axis=1,
                 keepdims=True) / denom
    var_b = jnp.sum(vt * present)

    cc = _dot(cn, cn, (((1,), (1,))))
    cd = jnp.sqrt(jnp.maximum(cn2 + jnp.transpose(cn2) - 2.0 * cc, 1e-12))
    eye = (jax.lax.broadcasted_iota(jnp.int32, (_C, _C), 0) ==
           jax.lax.broadcasted_iota(jnp.int32, (_C, _C), 1)).astype(_F32)
    pairmask = present * jnp.transpose(present) * (1.0 - eye)
    dist_num = jnp.sum(pairmask * jnp.maximum(2.0 * _DELTA_D - cd, 0.0))
    dist_b = dist_num / jnp.maximum(ni * (ni - 1.0), 1.0)

    reg_b = _GAMMA * jnp.sqrt(jnp.sum((cn * present) ** 2))

    sum_pts = _dot(masks, pts, (((1,), (1,))))
    gc = sum_pts / denom
    gp = _dot(gc, pts, (((1,), (0,))),
              precision=jax.lax.Precision.DEFAULT)
    p2 = jnp.sum(pts * pts, axis=0, keepdims=True)
    g2 = jnp.sum(gc * gc, axis=1, keepdims=True)
    sd = jnp.sqrt(jnp.maximum(p2 + g2 - 2.0 * gp, 0.0))
    fe = _dot(fc, emb, (((1,), (0,))),
              precision=jax.lax.Precision.DEFAULT)
    f2 = jnp.sum(fc * fc, axis=1, keepdims=True)
    fd = jnp.sqrt(jnp.maximum(en2 + f2 - 2.0 * fe, 0.0))
    ct = jnp.sum(masks * sd * fd, axis=1, keepdims=True) / denom
    center_b = jnp.sum(ct * present)

    mn = jnp.sqrt(jnp.sum(fc * fc, axis=1, keepdims=True))
    m_unit = fc / jnp.maximum(mn, 1e-12)
    mu_norm = jnp.sqrt(jnp.sum(m_unit * m_unit, axis=1, keepdims=True))
    sim = _dot(m_unit, emb, (((1,), (0,))))
    sim = sim / jnp.maximum(na * mu_norm, 1e-8)
    sim = (sim + 1.0) * 0.5

    emb_s[...] = emb
    en2_s[...] = en2
    en2T_s[...] = jnp.transpose(en2)
    labT_s[...] = jnp.transpose(lab)

    zero = _F32(0.0)

    def bouter(t, carry):
        r0 = t * _TILE
        etile = emb_s[:, pl.ds(r0, _TILE)]
        ej2 = en2T_s[pl.ds(r0, _TILE), :]
        labj = labT_s[pl.ds(r0, _TILE), :]

        def binner(u, c):
            pos_s, inst_s, dm_s = c
            c0 = u * _TILE
            ecol = emb_s[:, pl.ds(c0, _TILE)]
            dots = _dot(etile, ecol, (((0,), (0,))),
                        precision=jax.lax.Precision.DEFAULT)
            ek2 = en2_s[:, pl.ds(c0, _TILE)]
            dm = jnp.sqrt(jnp.maximum(ej2 + ek2 - 2.0 * dots, 1e-12))
            labk = lab_ref[0, :, pl.ds(c0, _TILE)]
            inst = (labj == labk).astype(_F32)
            w = jnp.where(u == t, _F32(1.0), _F32(2.0))
            pos_s = pos_s + w * jnp.sum(dm * inst)
            inst_s = inst_s + w * jnp.sum(inst)
            dm_s = dm_s + w * jnp.sum(dm)
            return pos_s, inst_s, dm_s

        return jax.lax.fori_loop(0, t + 1, binner, carry)

    pos_s, inst_s, dm_s = jax.lax.fori_loop(0, NT, bouter,
                                            (zero, zero, zero))
    bnd_cnt = _F32(N) * _F32(N) - inst_s
    neg_s = 2.0 * _DELTA_D * bnd_cnt - (dm_s - pos_s)
    boundary_b = (pos_s / jnp.maximum(inst_s, 1.0)
                  + neg_s / jnp.maximum(bnd_cnt, 1.0))

    lane = jax.lax.broadcasted_iota(jnp.int32, (1, N), 1)
    keys = (jax.lax.shift_left(
        jax.lax.bitcast_convert_type(jnp.maximum(sim, 0.0), jnp.int32), 1)
        | (lab == cid).astype(jnp.int32))
    k = 2
    while k <= N:
        desc = (lane & k) == 0
        j = k // 2
        while j >= 1:
            lo = (lane & j) == 0
            pk = jnp.where(lo, jnp.roll(keys, -j, axis=1),
                           jnp.roll(keys, j, axis=1))
            take_max = lo == desc
            keep = take_max == (keys > pk)
            keys = jnp.where(keep, keys, pk)
            j //= 2
        k *= 2
    payl = (keys & 1).astype(_F32)
    keys = jax.lax.bitcast_convert_type(
        jax.lax.shift_right_logical(keys, 1), _F32)

    cs = payl
    s = 1
    while s < N:
        cs = cs + jnp.where(lane >= s, jnp.roll(cs, s, axis=1), 0.0)
        s *= 2

    r = (lane + 1).astype(_F32)
    G = counts
    jac = 1.0 - (G - cs) / (G + r - cs + 1e-6)
    jacp = 1.0 - (G - cs + payl) / (G + r - 1.0 - cs + payl + 1e-6)
    lov_t = jnp.sum(jnp.maximum(keys, 0.0) * (jac - jacp), axis=1,
                    keepdims=True)
    lov_b = jnp.sum(lov_t * present) / jnp.maximum(ni, 1.0)

    vals = [var_b, dist_b, reg_b, center_b, boundary_b, lov_b, zero, zero]
    out_ref[0] = jnp.concatenate(
        [jnp.full((1, 128), v, _F32) for v in vals], axis=0)


def kernel(points, embeddings, instance_labels):
    B, N, _ = points.shape
    pts_t = jnp.moveaxis(jnp.pad(points, ((0, 0), (0, 0), (0, 5))), 2, 1)
    emb_t = jnp.moveaxis(embeddings, 2, 1)
    lab3 = instance_labels[:, None, :]

    out = pl.pallas_call(
        _loss_kernel,
        grid=(B,),
        in_specs=[
            pl.BlockSpec((1, emb_t.shape[1], N), lambda b: (b, 0, 0)),
            pl.BlockSpec((1, 8, N), lambda b: (b, 0, 0)),
            pl.BlockSpec((1, 1, N), lambda b: (b, 0, 0)),
        ],
        out_specs=pl.BlockSpec((1, 8, 128), lambda b: (b, 0, 0)),
        out_shape=jax.ShapeDtypeStruct((B, 8, 128), jnp.float32),
        compiler_params=pltpu.CompilerParams(
            dimension_semantics=("parallel",)),
        scratch_shapes=[
            pltpu.VMEM((emb_t.shape[1], N), jnp.float32),
            pltpu.VMEM((1, N), jnp.float32),
            pltpu.VMEM((N, 1), jnp.float32),
            pltpu.VMEM((N, 1), jnp.int32),
        ],
    )(emb_t, pts_t, lab3)

    v = out[:, :, 0]
    var_loss = jnp.sum(v[:, 0]) / (B + 1e-6)
    dist_loss = jnp.sum(v[:, 1]) / (B + 1e-6)
    reg_loss = jnp.sum(v[:, 2]) / (B + 1e-6)
    center_loss = jnp.sum(v[:, 3]) / B
    boundary_loss = jnp.sum(v[:, 4]) / B
    lovasz_loss = jnp.sum(v[:, 5]) / B
    total = (0.1 * (var_loss + dist_loss + reg_loss)
             + 0.1 * center_loss
             + 0.05 * boundary_loss
             + 0.01 * lovasz_loss)
    return (total, var_loss, dist_loss, reg_loss, center_loss,
            boundary_loss, lovasz_loss)


# speedup vs baseline: 1.2617x; 1.0861x over previous
import jax
import jax.numpy as jnp
from jax.experimental import pallas as pl
from jax.experimental.pallas import tpu as pltpu

_DELTA_V = 0.5
_DELTA_D = 1.5
_GAMMA = 0.001
_C = 24
_CL = 19
_TILE = 1024

_F32 = jnp.float32
_PREC = jax.lax.Precision.HIGHEST


def _dot(a, b, dims, precision=_PREC):
    return jax.lax.dot_general(a, b, (dims, ((), ())), precision=precision,
                               preferred_element_type=_F32)


def _loss_kernel(emb_t_ref, pts_t_ref, lab_ref, out_ref,
                 emb_s, en2_s, en2T_s, labT_s):
    embr = emb_t_ref[0]
    pts = pts_t_ref[0]
    lab = lab_ref[0]
    D, N = embr.shape
    NT = N // _TILE

    nrm = jnp.sqrt(jnp.sum(embr * embr, axis=0, keepdims=True))
    emb = embr / jnp.maximum(nrm, 1e-12)
    en2 = jnp.sum(emb * emb, axis=0, keepdims=True)
    na = jnp.sqrt(en2)

    cid = jax.lax.broadcasted_iota(jnp.int32, (_C, N), 0) + 1
    masks = (lab == cid).astype(_F32)
    counts = jnp.sum(masks, axis=1, keepdims=True)
    present = (counts > 0).astype(_F32)
    ni = jnp.sum(present)
    denom = jnp.maximum(counts, 1.0)

    sum_emb = _dot(masks, emb, (((1,), (1,))))
    fc = sum_emb / denom
    centers = fc * present
    cnorm = jnp.sqrt(jnp.sum(centers * centers, axis=1, keepdims=True))
    cn = centers / jnp.maximum(cnorm, 1e-12)

    ec = _dot(cn, emb, (((1,), (0,))),
              precision=jax.lax.Precision.DEFAULT)
    cn2 = jnp.sum(cn * cn, axis=1, keepdims=True)
    d = jnp.sqrt(jnp.maximum(en2 + cn2 - 2.0 * ec, 0.0))
    vt = jnp.sum(masks * jnp.maximum(d - _DELTA_V, 0.0), axis=1,
                 keepdims=True) / denom
    var_b = jnp.sum(vt * present)

    cc = _dot(cn, cn, (((1,), (1,))))
    cd = jnp.sqrt(jnp.maximum(cn2 + jnp.transpose(cn2) - 2.0 * cc, 1e-12))
    eye = (jax.lax.broadcasted_iota(jnp.int32, (_C, _C), 0) ==
           jax.lax.broadcasted_iota(jnp.int32, (_C, _C), 1)).astype(_F32)
    pairmask = present * jnp.transpose(present) * (1.0 - eye)
    dist_num = jnp.sum(pairmask * jnp.maximum(2.0 * _DELTA_D - cd, 0.0))
    dist_b = dist_num / jnp.maximum(ni * (ni - 1.0), 1.0)

    reg_b = _GAMMA * jnp.sqrt(jnp.sum((cn * present) ** 2))

    sum_pts = _dot(masks, pts, (((1,), (1,))))
    gc = sum_pts / denom
    gp = _dot(gc, pts, (((1,), (0,))),
              precision=jax.lax.Precision.DEFAULT)
    p2 = jnp.sum(pts * pts, axis=0, keepdims=True)
    g2 = jnp.sum(gc * gc, axis=1, keepdims=True)
    sd = jnp.sqrt(jnp.maximum(p2 + g2 - 2.0 * gp, 0.0))
    fe = _dot(fc, emb, (((1,), (0,))),
              precision=jax.lax.Precision.DEFAULT)
    f2 = jnp.sum(fc * fc, axis=1, keepdims=True)
    fd = jnp.sqrt(jnp.maximum(en2 + f2 - 2.0 * fe, 0.0))
    ct = jnp.sum(masks * sd * fd, axis=1, keepdims=True) / denom
    center_b = jnp.sum(ct * present)

    mn = jnp.sqrt(jnp.sum(fc * fc, axis=1, keepdims=True))
    m_unit = fc / jnp.maximum(mn, 1e-12)
    mu_norm = jnp.sqrt(jnp.sum(m_unit * m_unit, axis=1, keepdims=True))
    sim = _dot(m_unit, emb, (((1,), (0,))))
    sim = sim / jnp.maximum(na * mu_norm, 1e-8)
    sim = (sim + 1.0) * 0.5

    emb_s[...] = emb
    en2_s[...] = en2
    en2T_s[...] = jnp.transpose(en2)
    labT_s[...] = jnp.transpose(lab)

    zero = _F32(0.0)

    def bouter(t, carry):
        r0 = t * _TILE
        etile = emb_s[:, pl.ds(r0, _TILE)]
        ej2 = en2T_s[pl.ds(r0, _TILE), :]
        labj = labT_s[pl.ds(r0, _TILE), :]

        def binner(u, c):
            pos_s, inst_s, dm_s = c
            c0 = u * _TILE
            ecol = emb_s[:, pl.ds(c0, _TILE)]
            dots = _dot(etile, ecol, (((0,), (0,))),
                        precision=jax.lax.Precision.DEFAULT)
            ek2 = en2_s[:, pl.ds(c0, _TILE)]
            dm = jnp.sqrt(jnp.maximum(ej2 + ek2 - 2.0 * dots, 1e-12))
            labk = lab_ref[0, :, pl.ds(c0, _TILE)]
            inst = (labj == labk).astype(_F32)
            w = jnp.where(u == t, _F32(1.0), _F32(2.0))
            pos_s = pos_s + w * jnp.sum(dm * inst)
            inst_s = inst_s + w * jnp.sum(inst)
            dm_s = dm_s + w * jnp.sum(dm)
            return pos_s, inst_s, dm_s

        return jax.lax.fori_loop(0, t + 1, binner, carry)

    pos_s, inst_s, dm_s = jax.lax.fori_loop(0, NT, bouter,
                                            (zero, zero, zero))
    bnd_cnt = _F32(N) * _F32(N) - inst_s
    neg_s = 2.0 * _DELTA_D * bnd_cnt - (dm_s - pos_s)
    boundary_b = (pos_s / jnp.maximum(inst_s, 1.0)
                  + neg_s / jnp.maximum(bnd_cnt, 1.0))

    lane = jax.lax.broadcasted_iota(jnp.int32, (1, N), 1)
    keys = (jax.lax.shift_left(
        jax.lax.bitcast_convert_type(jnp.maximum(sim, 0.0), jnp.int32), 1)
        | (lab == cid).astype(jnp.int32))
    k = 2
    while k <= N:
        desc = (lane & k) == 0
        j = k // 2
        while j >= 1:
            lo = (lane & j) == 0
            pk = jnp.where(lo, jnp.roll(keys, -j, axis=1),
                           jnp.roll(keys, j, axis=1))
            take_max = lo == desc
            keep = take_max == (keys > pk)
            keys = jnp.where(keep, keys, pk)
            j //= 2
        k *= 2
    payl = (keys & 1).astype(_F32)
    keys = jax.lax.bitcast_convert_type(
        jax.lax.shift_right_logical(keys, 1), _F32)

    cs = payl
    s = 1
    while s < N:
        cs = cs + jnp.where(lane >= s, jnp.roll(cs, s, axis=1), 0.0)
        s *= 2

    r = (lane + 1).astype(_F32)
    G = counts
    jac = 1.0 - (G - cs) / (G + r - cs + 1e-6)
    jacp = 1.0 - (G - cs + payl) / (G + r - 1.0 - cs + payl + 1e-6)
    lov_t = jnp.sum(jnp.maximum(keys, 0.0) * (jac - jacp), axis=1,
                    keepdims=True)
    lov_b = jnp.sum(lov_t * present) / jnp.maximum(ni, 1.0)

    vals = [var_b, dist_b, reg_b, center_b, boundary_b, lov_b, zero, zero]
    out_ref[0] = jnp.concatenate(
        [jnp.full((1, 128), v, _F32) for v in vals], axis=0)


def kernel(points, embeddings, instance_labels):
    B, N, _ = points.shape
    pts_t = jnp.moveaxis(jnp.pad(points, ((0, 0), (0, 0), (0, 5))), 2, 1)
    emb_t = jnp.moveaxis(embeddings, 2, 1)
    lab3 = instance_labels[:, None, :]

    out = pl.pallas_call(
        _loss_kernel,
        grid=(B,),
        in_specs=[
            pl.BlockSpec((1, emb_t.shape[1], N), lambda b: (b, 0, 0)),
            pl.BlockSpec((1, 8, N), lambda b: (b, 0, 0)),
            pl.BlockSpec((1, 1, N), lambda b: (b, 0, 0)),
        ],
        out_specs=pl.BlockSpec((1, 8, 128), lambda b: (b, 0, 0)),
        out_shape=jax.ShapeDtypeStruct((B, 8, 128), jnp.float32),
        compiler_params=pltpu.CompilerParams(
            dimension_semantics=("parallel",)),
        scratch_shapes=[
            pltpu.VMEM((emb_t.shape[1], N), jnp.float32),
            pltpu.VMEM((1, N), jnp.float32),
            pltpu.VMEM((N, 1), jnp.float32),
            pltpu.VMEM((N, 1), jnp.int32),
        ],
    )(emb_t, pts_t, lab3)

    v = out[:, :, 0]
    var_loss = jnp.sum(v[:, 0]) / (B + 1e-6)
    dist_loss = jnp.sum(v[:, 1]) / (B + 1e-6)
    reg_loss = jnp.sum(v[:, 2]) / (B + 1e-6)
    center_loss = jnp.sum(v[:, 3]) / B
    boundary_loss = jnp.sum(v[:, 4]) / B
    lovasz_loss = jnp.sum(v[:, 5]) / B
    total = (0.1 * (var_loss + dist_loss + reg_loss)
             + 0.1 * center_loss
             + 0.05 * boundary_loss
             + 0.01 * lovasz_loss)
    return (total, var_loss, dist_loss, reg_loss, center_loss,
            boundary_loss, lovasz_loss)
